# baseline (speedup 1.0000x reference)
import functools

import jax
import jax.numpy as jnp
from jax import lax
from jax.experimental import pallas as pl
from jax.experimental.pallas import tpu as pltpu
from jax.experimental.pallas import tpu_sc as plsc

EMB = 64
MAXN = 50
MAXNP = 56
LANES = 128
CH = 256
CHB = CH // 128


def _sc_gather_all(neigh_idx, neighn_idx, users_idx, items_idx, itemsn_idx,
                   user_memory, item_memory, user_output):
    info = plsc.get_sparse_core_info()
    NC, NS = info.num_cores, info.num_subcores
    NW = NC * NS

    n_neigh = neigh_idx.shape[0] * CH
    n_small = users_idx.shape[0] * CH

    out_types = [
        jax.ShapeDtypeStruct((n_neigh, LANES), jnp.float32),
        jax.ShapeDtypeStruct((n_neigh, LANES), jnp.float32),
        jax.ShapeDtypeStruct((n_neigh, LANES), jnp.float32),
        jax.ShapeDtypeStruct((n_neigh, LANES), jnp.float32),
        jax.ShapeDtypeStruct((n_small, LANES), jnp.float32),
        jax.ShapeDtypeStruct((n_small, LANES), jnp.float32),
        jax.ShapeDtypeStruct((n_small, LANES), jnp.float32),
    ]
    mesh = plsc.VectorSubcoreMesh(core_axis_name="c", subcore_axis_name="s")

    @functools.partial(
        pl.kernel, mesh=mesh, out_type=out_types,
        compiler_params=pltpu.CompilerParams(use_tc_tiling_on_sc=False),
        scratch_types=[
            pltpu.VMEM((CHB, 128), jnp.int32),
            pltpu.VMEM((CHB, 128), jnp.int32),
            pltpu.VMEM((CH, LANES), jnp.float32),
            pltpu.VMEM((CH, LANES), jnp.float32),
            pltpu.SemaphoreType.DMA,
            pltpu.SemaphoreType.DMA,
            pltpu.SemaphoreType.DMA,
            pltpu.SemaphoreType.DMA,
        ],
    )
    def sc_kernel(neigh_ref, neighn_ref, users_ref, items_ref, itemsn_ref,
                  umem_ref, imem_ref, uout_ref,
                  nm_ref, no_ref, nmn_ref, non_ref, cu_ref, ci_ref, cin_ref,
                  idx0, idx1, rows0, rows1, g0, g1, o0, o1):
        wid = lax.axis_index("s") * NC + lax.axis_index("c")
        idx_b = (idx0, idx1)
        rows_b = (rows0, rows1)
        g_sem = (g0, g1)
        o_sem = (o0, o1)

        def start(idx_src, table, chunk, b):
            pltpu.sync_copy(idx_src.at[chunk], idx_b[b])
            for j in range(CHB):
                pltpu.async_copy(table.at[idx_b[b].at[j]],
                                 rows_b[b].at[pl.ds(j * 128, 128)], g_sem[b])

        def finish(table, out, chunk, b):
            for j in range(CHB):
                pltpu.make_async_copy(table.at[idx_b[b].at[j]],
                                      rows_b[b].at[pl.ds(j * 128, 128)],
                                      g_sem[b]).wait()
            pltpu.async_copy(rows_b[b], out.at[pl.ds(chunk * CH, CH)], o_sem[b])

        def drain(out, chunk, b):
            pltpu.make_async_copy(rows_b[b], out.at[pl.ds(chunk * CH, CH)],
                                  o_sem[b]).wait()

        def run_task(idx_src, table, out):
            n_chunks = idx_src.shape[0] // NW
            c_lo = wid * n_chunks
            if n_chunks == 1:
                start(idx_src, table, c_lo, 0)
                finish(table, out, c_lo, 0)
                drain(out, c_lo, 0)
                return
            n_half = n_chunks // 2

            start(idx_src, table, c_lo, 0)

            def body(c2, _):
                c = c_lo + 2 * c2
                finish(table, out, c, 0)
                start(idx_src, table, c + 1, 1)
                drain(out, c, 0)
                finish(table, out, c + 1, 1)

                @pl.when(c2 + 1 < n_half)
                def _():
                    start(idx_src, table, c + 2, 0)

                drain(out, c + 1, 1)
                return ()

            lax.fori_loop(0, n_half, body, ())

        run_task(neigh_ref, umem_ref, nm_ref)
        run_task(neigh_ref, uout_ref, no_ref)
        run_task(neighn_ref, umem_ref, nmn_ref)
        run_task(neighn_ref, uout_ref, non_ref)
        run_task(users_ref, umem_ref, cu_ref)
        run_task(items_ref, imem_ref, ci_ref)
        run_task(itemsn_ref, imem_ref, cin_ref)

    return sc_kernel(neigh_idx, neighn_idx, users_idx, items_idx, itemsn_idx,
                     user_memory, item_memory, user_output)


def _attn(nm, no, q, mask):
    scores = jnp.sum(nm * q[:, None, :], axis=-1)
    scores = jnp.where(mask, scores, jnp.finfo(scores.dtype).min)
    m = jnp.max(scores, axis=1, keepdims=True)
    e = jnp.exp(scores - m)
    p = e / jnp.sum(e, axis=1, keepdims=True)
    return jnp.sum(no * p[:, :, None], axis=1)


def _bf16_dot(a, b):
    return jnp.dot(a.astype(jnp.bfloat16), b.astype(jnp.bfloat16),
                   preferred_element_type=jnp.float32)


def _branch(u, v, nm, no, mask, hop_wT, hop_b, dense_wT, dense_b, out_w):
    q = u + v
    o0 = _attn(nm, no, q, mask)
    q1 = jax.nn.relu(_bf16_dot(q, hop_wT) + o0 + hop_b)
    o1 = _attn(nm, no, q1, mask)
    x = jnp.concatenate([u * v, o1], axis=1)
    h = jax.nn.relu(_bf16_dot(x, dense_wT) + dense_b)
    hb = h.astype(jnp.bfloat16).astype(jnp.float32)
    wb = out_w.astype(jnp.bfloat16).astype(jnp.float32)
    return jnp.sum(hb * wb, axis=1, keepdims=True)


def _cmn_kernel(bb, u_ref, v_ref, vn_ref, nm_ref, no_ref, nmn_ref, non_ref,
                len_ref, lenn_ref, hop_wT_ref, hop_b_ref, dense_wT_ref,
                dense_b_ref, out_w_ref, pos_ref, neg_ref):
    u = u_ref[...]
    hop_wT = hop_wT_ref[...]
    hop_b = hop_b_ref[...]
    dense_wT = dense_wT_ref[...]
    dense_b = dense_b_ref[...]
    out_w = out_w_ref[...]

    def neigh(ref):
        return ref[...].reshape(bb, MAXNP, LANES)

    pos_iota = jax.lax.broadcasted_iota(jnp.int32, (bb, MAXNP), 1)
    mask = pos_iota < len_ref[...]
    mask_n = pos_iota < lenn_ref[...]
    pos_ref[...] = _branch(u, v_ref[...], neigh(nm_ref), neigh(no_ref),
                           mask, hop_wT, hop_b, dense_wT, dense_b, out_w)
    neg_ref[...] = _branch(u, vn_ref[...], neigh(nmn_ref), neigh(non_ref),
                           mask_n, hop_wT, hop_b, dense_wT, dense_b, out_w)


def _pad_weights(hop_w, hop_b, dense_w, dense_b, out_w):
    hop_wT128 = jnp.pad(hop_w.T, ((0, LANES - EMB), (0, LANES - EMB)))
    hop_b128 = jnp.pad(hop_b.reshape(1, EMB), ((0, 0), (0, LANES - EMB)))
    dwT = dense_w.T
    dense_wT256 = jnp.pad(
        dwT.reshape(2, EMB, EMB), ((0, 0), (0, LANES - EMB), (0, LANES - EMB))
    ).reshape(2 * LANES, LANES)
    dense_b128 = jnp.pad(dense_b.reshape(1, EMB), ((0, 0), (0, LANES - EMB)))
    out_w128 = jnp.pad(out_w, ((0, 0), (0, LANES - EMB)))
    return hop_wT128, hop_b128, dense_wT256, dense_b128, out_w128


def _cmn_compute(cur_user, cur_item, cur_item_neg, nm, no, nmn, non,
                 lengths, lengths_n, hop_w, hop_b, dense_w, dense_b, out_w,
                 interpret=False):
    B = cur_user.shape[0]
    Bb = min(128, B)
    grid = (B // Bb,)
    row_spec = pl.BlockSpec((Bb, LANES), lambda i: (i, 0))
    neigh_spec = pl.BlockSpec((Bb * MAXNP, LANES), lambda i: (i, 0))
    len_spec = pl.BlockSpec((Bb, 1), lambda i: (i, 0))
    w_spec = pl.BlockSpec((LANES, LANES), lambda i: (0, 0))
    dw_spec = pl.BlockSpec((2 * LANES, LANES), lambda i: (0, 0))
    b_spec = pl.BlockSpec((1, LANES), lambda i: (0, 0))
    out_spec = pl.BlockSpec((Bb, 1), lambda i: (i, 0))
    hop_wT128, hop_b128, dense_wT256, dense_b128, out_w128 = _pad_weights(
        hop_w, hop_b, dense_w, dense_b, out_w)
    pos, neg = pl.pallas_call(
        functools.partial(_cmn_kernel, Bb),
        grid=grid,
        in_specs=[row_spec, row_spec, row_spec,
                  neigh_spec, neigh_spec, neigh_spec, neigh_spec,
                  len_spec, len_spec, w_spec, b_spec, dw_spec, b_spec, b_spec],
        out_specs=[out_spec, out_spec],
        out_shape=[jax.ShapeDtypeStruct((B, 1), jnp.float32),
                   jax.ShapeDtypeStruct((B, 1), jnp.float32)],
        interpret=interpret,
    )(cur_user, cur_item, cur_item_neg, nm, no, nmn, non,
      lengths.reshape(B, 1), lengths_n.reshape(B, 1),
      hop_wT128, hop_b128, dense_wT256, dense_b128, out_w128)
    return pos[:, 0], neg[:, 0]


def kernel(input_users, input_items, input_items_negative, input_neighborhoods,
           input_neighborhood_lengths, input_neighborhoods_negative,
           input_neighborhood_lengths_negative, user_memory, item_memory,
           user_output, hop_w, hop_b, dense_w, dense_b, out_w):
    B = input_users.shape[0]
    pad = ((0, 0), (0, LANES - EMB))
    um128 = jnp.pad(user_memory, pad)
    im128 = jnp.pad(item_memory, pad)
    uo128 = jnp.pad(user_output, pad)
    npad = ((0, 0), (0, MAXNP - MAXN))
    nidx = jnp.pad(input_neighborhoods, npad).reshape(-1, CHB, 128)
    nnidx = jnp.pad(input_neighborhoods_negative, npad).reshape(-1, CHB, 128)
    uidx = input_users.reshape(-1, CHB, 128)
    iidx = input_items.reshape(-1, CHB, 128)
    inidx = input_items_negative.reshape(-1, CHB, 128)
    nm, no, nmn, non, cu, ci, cin = _sc_gather_all(
        nidx, nnidx, uidx, iidx, inidx, um128, im128, uo128)
    return _cmn_compute(cu, ci, cin, nm, no, nmn, non,
                        input_neighborhood_lengths,
                        input_neighborhood_lengths_negative,
                        hop_w, hop_b, dense_w, dense_b, out_w)


# speedup vs baseline: 5.6085x; 5.6085x over previous
import functools

import jax
import jax.numpy as jnp
from jax import lax
from jax.experimental import pallas as pl
from jax.experimental.pallas import tpu as pltpu
from jax.experimental.pallas import tpu_sc as plsc

EMB = 64
MAXN = 50
MAXNP = 56
LANES = 128
CH = 256
CHB = CH // 128


def _sc_gather_all(neigh_idx, neighn_idx, users_idx, items_idx, itemsn_idx,
                   user_memory, item_memory, user_output):
    info = plsc.get_sparse_core_info()
    NC, NS = info.num_cores, info.num_subcores
    NW = NC * NS

    n_neigh = neigh_idx.shape[0] * CH
    n_small = users_idx.shape[0] * CH

    out_types = [
        jax.ShapeDtypeStruct((n_neigh, LANES), jnp.float32),
        jax.ShapeDtypeStruct((n_neigh, LANES), jnp.float32),
        jax.ShapeDtypeStruct((n_neigh, LANES), jnp.float32),
        jax.ShapeDtypeStruct((n_neigh, LANES), jnp.float32),
        jax.ShapeDtypeStruct((n_small, LANES), jnp.float32),
        jax.ShapeDtypeStruct((n_small, LANES), jnp.float32),
        jax.ShapeDtypeStruct((n_small, LANES), jnp.float32),
    ]
    mesh = plsc.VectorSubcoreMesh(core_axis_name="c", subcore_axis_name="s")

    @functools.partial(
        pl.kernel, mesh=mesh, out_type=out_types,
        compiler_params=pltpu.CompilerParams(use_tc_tiling_on_sc=False),
        scratch_types=[
            pltpu.VMEM((CHB, 128), jnp.int32),
            pltpu.VMEM((CHB, 128), jnp.int32),
            pltpu.VMEM((CH, LANES), jnp.float32),
            pltpu.VMEM((CH, LANES), jnp.float32),
            pltpu.SemaphoreType.DMA,
            pltpu.SemaphoreType.DMA,
            pltpu.SemaphoreType.DMA,
            pltpu.SemaphoreType.DMA,
        ],
    )
    def sc_kernel(neigh_ref, neighn_ref, users_ref, items_ref, itemsn_ref,
                  umem_ref, imem_ref, uout_ref,
                  nm_ref, no_ref, nmn_ref, non_ref, cu_ref, ci_ref, cin_ref,
                  idx0, idx1, rows0, rows1, g0, g1, o0, o1):
        wid = lax.axis_index("s") * NC + lax.axis_index("c")
        idx_b = (idx0, idx1)
        rows_b = (rows0, rows1)
        g_sem = (g0, g1)
        o_sem = (o0, o1)

        def start(idx_src, table, chunk, b):
            pltpu.sync_copy(idx_src.at[chunk], idx_b[b])
            for j in range(CHB):
                pltpu.async_copy(table.at[idx_b[b].at[j]],
                                 rows_b[b].at[pl.ds(j * 128, 128)], g_sem[b])

        def finish(table, out, chunk, b):
            for j in range(CHB):
                pltpu.make_async_copy(table.at[idx_b[b].at[j]],
                                      rows_b[b].at[pl.ds(j * 128, 128)],
                                      g_sem[b]).wait()
            pltpu.async_copy(rows_b[b], out.at[pl.ds(chunk * CH, CH)], o_sem[b])

        def drain(out, chunk, b):
            pltpu.make_async_copy(rows_b[b], out.at[pl.ds(chunk * CH, CH)],
                                  o_sem[b]).wait()

        def run_task(idx_src, table, out):
            n_chunks = idx_src.shape[0] // NW
            c_lo = wid * n_chunks
            if n_chunks == 1:
                start(idx_src, table, c_lo, 0)
                finish(table, out, c_lo, 0)
                drain(out, c_lo, 0)
                return
            n_half = n_chunks // 2

            start(idx_src, table, c_lo, 0)

            def body(c2, _):
                c = c_lo + 2 * c2
                finish(table, out, c, 0)
                start(idx_src, table, c + 1, 1)
                drain(out, c, 0)
                finish(table, out, c + 1, 1)

                @pl.when(c2 + 1 < n_half)
                def _():
                    start(idx_src, table, c + 2, 0)

                drain(out, c + 1, 1)
                return ()

            lax.fori_loop(0, n_half, body, ())

        run_task(neigh_ref, umem_ref, nm_ref)
        run_task(neigh_ref, uout_ref, no_ref)
        run_task(neighn_ref, umem_ref, nmn_ref)
        run_task(neighn_ref, uout_ref, non_ref)
        run_task(users_ref, umem_ref, cu_ref)
        run_task(items_ref, imem_ref, ci_ref)
        run_task(itemsn_ref, imem_ref, cin_ref)

    return sc_kernel(neigh_idx, neighn_idx, users_idx, items_idx, itemsn_idx,
                     user_memory, item_memory, user_output)


def _attn(nm, no, q, mask):
    scores = jnp.sum(nm * q[:, None, :], axis=-1)
    scores = jnp.where(mask, scores, jnp.finfo(scores.dtype).min)
    m = jnp.max(scores, axis=1, keepdims=True)
    e = jnp.exp(scores - m)
    p = e / jnp.sum(e, axis=1, keepdims=True)
    return jnp.sum(no * p[:, :, None], axis=1)


def _bf16_dot(a, b):
    return jnp.dot(a.astype(jnp.bfloat16), b.astype(jnp.bfloat16),
                   preferred_element_type=jnp.float32)


def _branch(u, v, nm, no, mask, hop_wT, hop_b, dense_wT, dense_b, out_w):
    q = u + v
    o0 = _attn(nm, no, q, mask)
    q1 = jax.nn.relu(_bf16_dot(q, hop_wT) + o0 + hop_b)
    o1 = _attn(nm, no, q1, mask)
    x = jnp.concatenate([u * v, o1], axis=1)
    h = jax.nn.relu(_bf16_dot(x, dense_wT) + dense_b)
    hb = h.astype(jnp.bfloat16).astype(jnp.float32)
    wb = out_w.astype(jnp.bfloat16).astype(jnp.float32)
    return jnp.sum(hb * wb, axis=1, keepdims=True)


def _cmn_kernel(bb, u_ref, v_ref, vn_ref, nm_ref, no_ref, nmn_ref, non_ref,
                len_ref, lenn_ref, hop_wT_ref, hop_b_ref, dense_wT_ref,
                dense_b_ref, out_w_ref, pos_ref, neg_ref):
    u = u_ref[...]
    hop_wT = hop_wT_ref[...]
    hop_b = hop_b_ref[...]
    dense_wT = dense_wT_ref[...]
    dense_b = dense_b_ref[...]
    out_w = out_w_ref[...]

    def neigh(ref):
        return ref[...].reshape(bb, MAXNP, LANES)

    pos_iota = jax.lax.broadcasted_iota(jnp.int32, (bb, MAXNP), 1)
    mask = pos_iota < len_ref[...]
    mask_n = pos_iota < lenn_ref[...]
    pos_ref[...] = _branch(u, v_ref[...], neigh(nm_ref), neigh(no_ref),
                           mask, hop_wT, hop_b, dense_wT, dense_b, out_w)
    neg_ref[...] = _branch(u, vn_ref[...], neigh(nmn_ref), neigh(non_ref),
                           mask_n, hop_wT, hop_b, dense_wT, dense_b, out_w)


def _pad_weights(hop_w, hop_b, dense_w, dense_b, out_w):
    hop_wT128 = jnp.pad(hop_w.T, ((0, LANES - EMB), (0, LANES - EMB)))
    hop_b128 = jnp.pad(hop_b.reshape(1, EMB), ((0, 0), (0, LANES - EMB)))
    dwT = dense_w.T
    dense_wT256 = jnp.pad(
        dwT.reshape(2, EMB, EMB), ((0, 0), (0, LANES - EMB), (0, LANES - EMB))
    ).reshape(2 * LANES, LANES)
    dense_b128 = jnp.pad(dense_b.reshape(1, EMB), ((0, 0), (0, LANES - EMB)))
    out_w128 = jnp.pad(out_w, ((0, 0), (0, LANES - EMB)))
    return hop_wT128, hop_b128, dense_wT256, dense_b128, out_w128


def _cmn_compute(cur_user, cur_item, cur_item_neg, nm, no, nmn, non,
                 lengths, lengths_n, hop_w, hop_b, dense_w, dense_b, out_w,
                 interpret=False):
    B = cur_user.shape[0]
    Bb = min(128, B)
    grid = (B // Bb,)
    row_spec = pl.BlockSpec((Bb, LANES), lambda i: (i, 0))
    neigh_spec = pl.BlockSpec((Bb * MAXNP, LANES), lambda i: (i, 0))
    len_spec = pl.BlockSpec((Bb, 1), lambda i: (i, 0))
    w_spec = pl.BlockSpec((LANES, LANES), lambda i: (0, 0))
    dw_spec = pl.BlockSpec((2 * LANES, LANES), lambda i: (0, 0))
    b_spec = pl.BlockSpec((1, LANES), lambda i: (0, 0))
    out_spec = pl.BlockSpec((Bb, 1), lambda i: (i, 0))
    hop_wT128, hop_b128, dense_wT256, dense_b128, out_w128 = _pad_weights(
        hop_w, hop_b, dense_w, dense_b, out_w)
    pos, neg = pl.pallas_call(
        functools.partial(_cmn_kernel, Bb),
        grid=grid,
        in_specs=[row_spec, row_spec, row_spec,
                  neigh_spec, neigh_spec, neigh_spec, neigh_spec,
                  len_spec, len_spec, w_spec, b_spec, dw_spec, b_spec, b_spec],
        out_specs=[out_spec, out_spec],
        out_shape=[jax.ShapeDtypeStruct((B, 1), jnp.float32),
                   jax.ShapeDtypeStruct((B, 1), jnp.float32)],
        interpret=interpret,
    )(cur_user, cur_item, cur_item_neg, nm, no, nmn, non,
      lengths.reshape(B, 1), lengths_n.reshape(B, 1),
      hop_wT128, hop_b128, dense_wT256, dense_b128, out_w128)
    return pos[:, 0], neg[:, 0]


def kernel(input_users, input_items, input_items_negative, input_neighborhoods,
           input_neighborhood_lengths, input_neighborhoods_negative,
           input_neighborhood_lengths_negative, user_memory, item_memory,
           user_output, hop_w, hop_b, dense_w, dense_b, out_w):
    B = input_users.shape[0]
    pad = ((0, 0), (0, LANES - EMB))
    um128 = jnp.pad(user_memory, pad)
    im128 = jnp.pad(item_memory, pad)
    uo128 = jnp.pad(user_output, pad)
    dummy = jnp.arange(B * (MAXNP - MAXN), dtype=jnp.int32).reshape(
        B, MAXNP - MAXN) % user_memory.shape[0]
    nidx = jnp.concatenate([input_neighborhoods, dummy], axis=1
                           ).reshape(-1, CHB, 128)
    nnidx = jnp.concatenate([input_neighborhoods_negative, dummy], axis=1
                            ).reshape(-1, CHB, 128)
    uidx = input_users.reshape(-1, CHB, 128)
    iidx = input_items.reshape(-1, CHB, 128)
    inidx = input_items_negative.reshape(-1, CHB, 128)
    nm, no, nmn, non, cu, ci, cin = _sc_gather_all(
        nidx, nnidx, uidx, iidx, inidx, um128, im128, uo128)
    return _cmn_compute(cu, ci, cin, nm, no, nmn, non,
                        input_neighborhood_lengths,
                        input_neighborhood_lengths_negative,
                        hop_w, hop_b, dense_w, dense_b, out_w)


# speedup vs baseline: 6.1481x; 1.0962x over previous
import functools

import jax
import jax.numpy as jnp
from jax import lax
from jax.experimental import pallas as pl
from jax.experimental.pallas import tpu as pltpu
from jax.experimental.pallas import tpu_sc as plsc

EMB = 64
MAXN = 50
MAXNP = 56
LANES = 128
CH = 256
CHB = CH // 128


def _sc_gather_all(neigh_idx, neighn_idx, users_idx, items_idx, itemsn_idx,
                   user_memory, item_memory, user_output):
    info = plsc.get_sparse_core_info()
    NC, NS = info.num_cores, info.num_subcores
    NW = NC * NS

    n_neigh = neigh_idx.shape[0] * CH
    n_small = users_idx.shape[0] * CH

    out_types = [
        jax.ShapeDtypeStruct((n_neigh, LANES), jnp.float32),
        jax.ShapeDtypeStruct((n_neigh, LANES), jnp.float32),
        jax.ShapeDtypeStruct((n_neigh, LANES), jnp.float32),
        jax.ShapeDtypeStruct((n_neigh, LANES), jnp.float32),
        jax.ShapeDtypeStruct((n_small, LANES), jnp.float32),
        jax.ShapeDtypeStruct((n_small, LANES), jnp.float32),
        jax.ShapeDtypeStruct((n_small, LANES), jnp.float32),
    ]
    mesh = plsc.VectorSubcoreMesh(core_axis_name="c", subcore_axis_name="s")

    @functools.partial(
        pl.kernel, mesh=mesh, out_type=out_types,
        compiler_params=pltpu.CompilerParams(use_tc_tiling_on_sc=False),
        scratch_types=[
            pltpu.VMEM((CHB, 128), jnp.int32),
            pltpu.VMEM((CHB, 128), jnp.int32),
            pltpu.VMEM((CH, LANES), jnp.float32),
            pltpu.VMEM((CH, LANES), jnp.float32),
            pltpu.SemaphoreType.DMA,
            pltpu.SemaphoreType.DMA,
            pltpu.SemaphoreType.DMA,
            pltpu.SemaphoreType.DMA,
        ],
    )
    def sc_kernel(neigh_ref, neighn_ref, users_ref, items_ref, itemsn_ref,
                  umem_ref, imem_ref, uout_ref,
                  nm_ref, no_ref, nmn_ref, non_ref, cu_ref, ci_ref, cin_ref,
                  idx0, idx1, rows0, rows1, g0, g1, o0, o1):
        wid = lax.axis_index("s") * NC + lax.axis_index("c")
        idx_b = (idx0, idx1)
        rows_b = (rows0, rows1)
        g_sem = (g0, g1)
        o_sem = (o0, o1)

        def start(idx_src, table, chunk, b):
            pltpu.sync_copy(idx_src.at[chunk], idx_b[b])
            for j in range(CHB):
                pltpu.async_copy(table.at[idx_b[b].at[j]],
                                 rows_b[b].at[pl.ds(j * 128, 128)], g_sem[b])

        def finish(table, out, chunk, b):
            for j in range(CHB):
                pltpu.make_async_copy(table.at[idx_b[b].at[j]],
                                      rows_b[b].at[pl.ds(j * 128, 128)],
                                      g_sem[b]).wait()
            pltpu.async_copy(rows_b[b], out.at[pl.ds(chunk * CH, CH)], o_sem[b])

        def drain(out, chunk, b):
            pltpu.make_async_copy(rows_b[b], out.at[pl.ds(chunk * CH, CH)],
                                  o_sem[b]).wait()

        def run_task(idx_src, table, out):
            n_chunks = idx_src.shape[0] // NW
            c_lo = wid * n_chunks
            if n_chunks == 1:
                start(idx_src, table, c_lo, 0)
                finish(table, out, c_lo, 0)
                drain(out, c_lo, 0)
                return
            n_half = n_chunks // 2

            start(idx_src, table, c_lo, 0)

            def body(c2, _):
                c = c_lo + 2 * c2
                finish(table, out, c, 0)
                start(idx_src, table, c + 1, 1)
                drain(out, c, 0)
                finish(table, out, c + 1, 1)

                @pl.when(c2 + 1 < n_half)
                def _():
                    start(idx_src, table, c + 2, 0)

                drain(out, c + 1, 1)
                return ()

            lax.fori_loop(0, n_half, body, ())

        run_task(neigh_ref, umem_ref, nm_ref)
        run_task(neigh_ref, uout_ref, no_ref)
        run_task(neighn_ref, umem_ref, nmn_ref)
        run_task(neighn_ref, uout_ref, non_ref)
        run_task(users_ref, umem_ref, cu_ref)
        run_task(items_ref, imem_ref, ci_ref)
        run_task(itemsn_ref, imem_ref, cin_ref)

    return sc_kernel(neigh_idx, neighn_idx, users_idx, items_idx, itemsn_idx,
                     user_memory, item_memory, user_output)


def _attn(nm, no, q, mask):
    scores = jnp.sum(nm * q[:, None, :], axis=-1)
    scores = jnp.where(mask, scores, jnp.finfo(scores.dtype).min)
    m = jnp.max(scores, axis=1, keepdims=True)
    e = jnp.exp(scores - m)
    p = e / jnp.sum(e, axis=1, keepdims=True)
    return jnp.sum(no * p[:, :, None], axis=1)


def _bf16_dot(a, b):
    return jnp.dot(a.astype(jnp.bfloat16), b.astype(jnp.bfloat16),
                   preferred_element_type=jnp.float32)


def _branch(u, v, nm, no, mask, hop_wT, hop_b, dense_wT, dense_b, out_w):
    q = u + v
    o0 = _attn(nm, no, q, mask)
    q1 = jax.nn.relu(_bf16_dot(q, hop_wT) + o0 + hop_b)
    o1 = _attn(nm, no, q1, mask)
    x = jnp.concatenate([u * v, o1], axis=1)
    h = jax.nn.relu(_bf16_dot(x, dense_wT) + dense_b)
    hb = h.astype(jnp.bfloat16).astype(jnp.float32)
    wb = out_w.astype(jnp.bfloat16).astype(jnp.float32)
    return jnp.sum(hb * wb, axis=1, keepdims=True)


def _cmn_kernel(bb, u_ref, v_ref, vn_ref, nm_ref, no_ref, nmn_ref, non_ref,
                len_ref, lenn_ref, hop_wT_ref, hop_b_ref, dense_wT_ref,
                dense_b_ref, out_w_ref, pos_ref, neg_ref):
    u = u_ref[...]
    hop_wT = hop_wT_ref[...]
    hop_b = hop_b_ref[...]
    dense_wT = dense_wT_ref[...]
    dense_b = dense_b_ref[...]
    out_w = out_w_ref[...]

    def neigh(ref):
        return ref[...].reshape(bb, MAXNP, LANES)

    pos_iota = jax.lax.broadcasted_iota(jnp.int32, (bb, MAXNP), 1)
    mask = pos_iota < len_ref[...]
    mask_n = pos_iota < lenn_ref[...]
    pos_ref[...] = _branch(u, v_ref[...], neigh(nm_ref), neigh(no_ref),
                           mask, hop_wT, hop_b, dense_wT, dense_b, out_w)
    neg_ref[...] = _branch(u, vn_ref[...], neigh(nmn_ref), neigh(non_ref),
                           mask_n, hop_wT, hop_b, dense_wT, dense_b, out_w)


def _pad_weights(hop_w, hop_b, dense_w, dense_b, out_w):
    hop_wT128 = jnp.pad(hop_w.T, ((0, LANES - EMB), (0, LANES - EMB)))
    hop_b128 = jnp.pad(hop_b.reshape(1, EMB), ((0, 0), (0, LANES - EMB)))
    dwT = dense_w.T
    dense_wT256 = jnp.pad(
        dwT.reshape(2, EMB, EMB), ((0, 0), (0, LANES - EMB), (0, LANES - EMB))
    ).reshape(2 * LANES, LANES)
    dense_b128 = jnp.pad(dense_b.reshape(1, EMB), ((0, 0), (0, LANES - EMB)))
    out_w128 = jnp.pad(out_w, ((0, 0), (0, LANES - EMB)))
    return hop_wT128, hop_b128, dense_wT256, dense_b128, out_w128


def _cmn_compute(cur_user, cur_item, cur_item_neg, nm, no, nmn, non,
                 lengths, lengths_n, hop_w, hop_b, dense_w, dense_b, out_w,
                 interpret=False):
    B = cur_user.shape[0]
    Bb = min(128, B)
    grid = (B // Bb,)
    row_spec = pl.BlockSpec((Bb, LANES), lambda i: (i, 0))
    neigh_spec = pl.BlockSpec((Bb * MAXNP, LANES), lambda i: (i, 0))
    len_spec = pl.BlockSpec((Bb, 1), lambda i: (i, 0))
    w_spec = pl.BlockSpec((LANES, LANES), lambda i: (0, 0))
    dw_spec = pl.BlockSpec((2 * LANES, LANES), lambda i: (0, 0))
    b_spec = pl.BlockSpec((1, LANES), lambda i: (0, 0))
    out_spec = pl.BlockSpec((Bb, 1), lambda i: (i, 0))
    hop_wT128, hop_b128, dense_wT256, dense_b128, out_w128 = _pad_weights(
        hop_w, hop_b, dense_w, dense_b, out_w)
    pos, neg = pl.pallas_call(
        functools.partial(_cmn_kernel, Bb),
        grid=grid,
        in_specs=[row_spec, row_spec, row_spec,
                  neigh_spec, neigh_spec, neigh_spec, neigh_spec,
                  len_spec, len_spec, w_spec, b_spec, dw_spec, b_spec, b_spec],
        out_specs=[out_spec, out_spec],
        out_shape=[jax.ShapeDtypeStruct((B, 1), jnp.float32),
                   jax.ShapeDtypeStruct((B, 1), jnp.float32)],
        interpret=interpret,
    )(cur_user, cur_item, cur_item_neg, nm, no, nmn, non,
      lengths.reshape(B, 1), lengths_n.reshape(B, 1),
      hop_wT128, hop_b128, dense_wT256, dense_b128, out_w128)
    return pos[:, 0], neg[:, 0]


NSLICES = 2


def kernel(input_users, input_items, input_items_negative, input_neighborhoods,
           input_neighborhood_lengths, input_neighborhoods_negative,
           input_neighborhood_lengths_negative, user_memory, item_memory,
           user_output, hop_w, hop_b, dense_w, dense_b, out_w):
    B = input_users.shape[0]
    pad = ((0, 0), (0, LANES - EMB))
    um128 = jnp.pad(user_memory, pad)
    im128 = jnp.pad(item_memory, pad)
    uo128 = jnp.pad(user_output, pad)
    Bs = B // NSLICES
    dummy = jnp.arange(Bs * (MAXNP - MAXN), dtype=jnp.int32).reshape(
        Bs, MAXNP - MAXN) % user_memory.shape[0]

    pos_parts, neg_parts = [], []
    for s in range(NSLICES):
        sl = slice(s * Bs, (s + 1) * Bs)
        nidx = jnp.concatenate([input_neighborhoods[sl], dummy], axis=1
                               ).reshape(-1, CHB, 128)
        nnidx = jnp.concatenate([input_neighborhoods_negative[sl], dummy],
                                axis=1).reshape(-1, CHB, 128)
        uidx = input_users[sl].reshape(-1, CHB, 128)
        iidx = input_items[sl].reshape(-1, CHB, 128)
        inidx = input_items_negative[sl].reshape(-1, CHB, 128)
        nm, no, nmn, non, cu, ci, cin = _sc_gather_all(
            nidx, nnidx, uidx, iidx, inidx, um128, im128, uo128)
        pos, neg = _cmn_compute(cu, ci, cin, nm, no, nmn, non,
                                input_neighborhood_lengths[sl],
                                input_neighborhood_lengths_negative[sl],
                                hop_w, hop_b, dense_w, dense_b, out_w)
        pos_parts.append(pos)
        neg_parts.append(neg)
    if NSLICES == 1:
        return pos_parts[0], neg_parts[0]
    return (jnp.concatenate(pos_parts), jnp.concatenate(neg_parts))


# speedup vs baseline: 6.2456x; 1.0159x over previous
import functools

import jax
import jax.numpy as jnp
from jax import lax
from jax.experimental import pallas as pl
from jax.experimental.pallas import tpu as pltpu
from jax.experimental.pallas import tpu_sc as plsc

EMB = 64
MAXN = 50
MAXNP = 56
LANES = 128
CH = 256
CHB = CH // 128


def _sc_gather_from(table, *idx_arrays):
    info = plsc.get_sparse_core_info()
    NC, NS = info.num_cores, info.num_subcores
    NW = NC * NS

    out_types = [jax.ShapeDtypeStruct((ia.shape[0] * CH, LANES), jnp.float32)
                 for ia in idx_arrays]
    mesh = plsc.VectorSubcoreMesh(core_axis_name="c", subcore_axis_name="s")

    @functools.partial(
        pl.kernel, mesh=mesh, out_type=out_types,
        compiler_params=pltpu.CompilerParams(use_tc_tiling_on_sc=False),
        scratch_types=[
            pltpu.VMEM((CHB, 128), jnp.int32),
            pltpu.VMEM((CHB, 128), jnp.int32),
            pltpu.VMEM((CH, LANES), jnp.float32),
            pltpu.VMEM((CH, LANES), jnp.float32),
            pltpu.SemaphoreType.DMA,
            pltpu.SemaphoreType.DMA,
            pltpu.SemaphoreType.DMA,
            pltpu.SemaphoreType.DMA,
        ],
    )
    def sc_kernel(table_ref, *refs):
        n_idx = len(idx_arrays)
        idx_refs = refs[:n_idx]
        out_refs = refs[n_idx:2 * n_idx]
        idx0, idx1, rows0, rows1, g0, g1, o0, o1 = refs[2 * n_idx:]
        wid = lax.axis_index("s") * NC + lax.axis_index("c")
        idx_b = (idx0, idx1)
        rows_b = (rows0, rows1)
        g_sem = (g0, g1)
        o_sem = (o0, o1)

        def start(idx_src, table, chunk, b):
            pltpu.sync_copy(idx_src.at[chunk], idx_b[b])
            for j in range(CHB):
                pltpu.async_copy(table.at[idx_b[b].at[j]],
                                 rows_b[b].at[pl.ds(j * 128, 128)], g_sem[b])

        def finish(table, out, chunk, b):
            for j in range(CHB):
                pltpu.make_async_copy(table.at[idx_b[b].at[j]],
                                      rows_b[b].at[pl.ds(j * 128, 128)],
                                      g_sem[b]).wait()
            pltpu.async_copy(rows_b[b], out.at[pl.ds(chunk * CH, CH)], o_sem[b])

        def drain(out, chunk, b):
            pltpu.make_async_copy(rows_b[b], out.at[pl.ds(chunk * CH, CH)],
                                  o_sem[b]).wait()

        def run_task(idx_src, table, out):
            n_chunks = idx_src.shape[0] // NW
            c_lo = wid * n_chunks
            if n_chunks == 1:
                start(idx_src, table, c_lo, 0)
                finish(table, out, c_lo, 0)
                drain(out, c_lo, 0)
                return
            n_half = n_chunks // 2

            start(idx_src, table, c_lo, 0)

            def body(c2, _):
                c = c_lo + 2 * c2
                finish(table, out, c, 0)
                start(idx_src, table, c + 1, 1)
                drain(out, c, 0)
                finish(table, out, c + 1, 1)

                @pl.when(c2 + 1 < n_half)
                def _():
                    start(idx_src, table, c + 2, 0)

                drain(out, c + 1, 1)
                return ()

            lax.fori_loop(0, n_half, body, ())

        for idx_ref, out_ref in zip(idx_refs, out_refs):
            run_task(idx_ref, table_ref, out_ref)

    outs = sc_kernel(table, *idx_arrays)
    return outs if isinstance(outs, (tuple, list)) else (outs,)


def _attn(nm, no, q, mask):
    scores = jnp.sum(nm * q[:, None, :], axis=-1)
    scores = jnp.where(mask, scores, jnp.finfo(scores.dtype).min)
    m = jnp.max(scores, axis=1, keepdims=True)
    e = jnp.exp(scores - m)
    p = e / jnp.sum(e, axis=1, keepdims=True)
    return jnp.sum(no * p[:, :, None], axis=1)


def _bf16_dot(a, b):
    return jnp.dot(a.astype(jnp.bfloat16), b.astype(jnp.bfloat16),
                   preferred_element_type=jnp.float32)


def _branch(u, v, nm, no, mask, hop_wT, hop_b, dense_wT, dense_b, out_w):
    q = u + v
    o0 = _attn(nm, no, q, mask)
    q1 = jax.nn.relu(_bf16_dot(q, hop_wT) + o0 + hop_b)
    o1 = _attn(nm, no, q1, mask)
    x = jnp.concatenate([u * v, o1], axis=1)
    h = jax.nn.relu(_bf16_dot(x, dense_wT) + dense_b)
    hb = h.astype(jnp.bfloat16).astype(jnp.float32)
    wb = out_w.astype(jnp.bfloat16).astype(jnp.float32)
    return jnp.sum(hb * wb, axis=1, keepdims=True)


def _cmn_kernel(bb, u_ref, v_ref, vn_ref, nm_ref, no_ref, nmn_ref, non_ref,
                len_ref, lenn_ref, hop_wT_ref, hop_b_ref, dense_wT_ref,
                dense_b_ref, out_w_ref, pos_ref, neg_ref):
    u = u_ref[...]
    hop_wT = hop_wT_ref[...]
    hop_b = hop_b_ref[...]
    dense_wT = dense_wT_ref[...]
    dense_b = dense_b_ref[...]
    out_w = out_w_ref[...]

    def neigh(ref):
        return ref[...].reshape(bb, MAXNP, LANES)

    pos_iota = jax.lax.broadcasted_iota(jnp.int32, (bb, MAXNP), 1)
    mask = pos_iota < len_ref[...]
    mask_n = pos_iota < lenn_ref[...]
    pos_ref[...] = _branch(u, v_ref[...], neigh(nm_ref), neigh(no_ref),
                           mask, hop_wT, hop_b, dense_wT, dense_b, out_w)
    neg_ref[...] = _branch(u, vn_ref[...], neigh(nmn_ref), neigh(non_ref),
                           mask_n, hop_wT, hop_b, dense_wT, dense_b, out_w)


def _pad_weights(hop_w, hop_b, dense_w, dense_b, out_w):
    hop_wT128 = jnp.pad(hop_w.T, ((0, LANES - EMB), (0, LANES - EMB)))
    hop_b128 = jnp.pad(hop_b.reshape(1, EMB), ((0, 0), (0, LANES - EMB)))
    dwT = dense_w.T
    dense_wT256 = jnp.pad(
        dwT.reshape(2, EMB, EMB), ((0, 0), (0, LANES - EMB), (0, LANES - EMB))
    ).reshape(2 * LANES, LANES)
    dense_b128 = jnp.pad(dense_b.reshape(1, EMB), ((0, 0), (0, LANES - EMB)))
    out_w128 = jnp.pad(out_w, ((0, 0), (0, LANES - EMB)))
    return hop_wT128, hop_b128, dense_wT256, dense_b128, out_w128


def _cmn_compute(cur_user, cur_item, cur_item_neg, nm, no, nmn, non,
                 lengths, lengths_n, hop_w, hop_b, dense_w, dense_b, out_w,
                 interpret=False):
    B = cur_user.shape[0]
    Bb = min(128, B)
    grid = (B // Bb,)
    row_spec = pl.BlockSpec((Bb, LANES), lambda i: (i, 0))
    neigh_spec = pl.BlockSpec((Bb * MAXNP, LANES), lambda i: (i, 0))
    len_spec = pl.BlockSpec((Bb, 1), lambda i: (i, 0))
    w_spec = pl.BlockSpec((LANES, LANES), lambda i: (0, 0))
    dw_spec = pl.BlockSpec((2 * LANES, LANES), lambda i: (0, 0))
    b_spec = pl.BlockSpec((1, LANES), lambda i: (0, 0))
    out_spec = pl.BlockSpec((Bb, 1), lambda i: (i, 0))
    hop_wT128, hop_b128, dense_wT256, dense_b128, out_w128 = _pad_weights(
        hop_w, hop_b, dense_w, dense_b, out_w)
    pos, neg = pl.pallas_call(
        functools.partial(_cmn_kernel, Bb),
        grid=grid,
        in_specs=[row_spec, row_spec, row_spec,
                  neigh_spec, neigh_spec, neigh_spec, neigh_spec,
                  len_spec, len_spec, w_spec, b_spec, dw_spec, b_spec, b_spec],
        out_specs=[out_spec, out_spec],
        out_shape=[jax.ShapeDtypeStruct((B, 1), jnp.float32),
                   jax.ShapeDtypeStruct((B, 1), jnp.float32)],
        interpret=interpret,
    )(cur_user, cur_item, cur_item_neg, nm, no, nmn, non,
      lengths.reshape(B, 1), lengths_n.reshape(B, 1),
      hop_wT128, hop_b128, dense_wT256, dense_b128, out_w128)
    return pos[:, 0], neg[:, 0]


NSLICES = 2


def kernel(input_users, input_items, input_items_negative, input_neighborhoods,
           input_neighborhood_lengths, input_neighborhoods_negative,
           input_neighborhood_lengths_negative, user_memory, item_memory,
           user_output, hop_w, hop_b, dense_w, dense_b, out_w):
    B = input_users.shape[0]
    pad = ((0, 0), (0, LANES - EMB))
    um128 = jnp.pad(user_memory, pad)
    im128 = jnp.pad(item_memory, pad)
    uo128 = jnp.pad(user_output, pad)
    Bs = B // NSLICES
    dummy = jnp.arange(Bs * (MAXNP - MAXN), dtype=jnp.int32).reshape(
        Bs, MAXNP - MAXN) % user_memory.shape[0]

    pos_parts, neg_parts = [], []
    for s in range(NSLICES):
        sl = slice(s * Bs, (s + 1) * Bs)
        nidx = jnp.concatenate([input_neighborhoods[sl], dummy], axis=1
                               ).reshape(-1, CHB, 128)
        nnidx = jnp.concatenate([input_neighborhoods_negative[sl], dummy],
                                axis=1).reshape(-1, CHB, 128)
        uidx = input_users[sl].reshape(-1, CHB, 128)
        iidx = input_items[sl].reshape(-1, CHB, 128)
        inidx = input_items_negative[sl].reshape(-1, CHB, 128)
        ci, cin = _sc_gather_from(im128, iidx, inidx)
        nm, nmn, cu = _sc_gather_from(um128, nidx, nnidx, uidx)
        no, non = _sc_gather_from(uo128, nidx, nnidx)
        pos, neg = _cmn_compute(cu, ci, cin, nm, no, nmn, non,
                                input_neighborhood_lengths[sl],
                                input_neighborhood_lengths_negative[sl],
                                hop_w, hop_b, dense_w, dense_b, out_w)
        pos_parts.append(pos)
        neg_parts.append(neg)
    if NSLICES == 1:
        return pos_parts[0], neg_parts[0]
    return (jnp.concatenate(pos_parts), jnp.concatenate(neg_parts))


# speedup vs baseline: 6.4638x; 1.0349x over previous
import functools

import jax
import jax.numpy as jnp
from jax import lax
from jax.experimental import pallas as pl
from jax.experimental.pallas import tpu as pltpu
from jax.experimental.pallas import tpu_sc as plsc

EMB = 64
MAXN = 50
MAXNP = 56
LANES = 128
CH = 256
CHB = CH // 128


def _sc_gather_from(table, *idx_arrays):
    info = plsc.get_sparse_core_info()
    NC, NS = info.num_cores, info.num_subcores
    NW = NC * NS

    out_types = [jax.ShapeDtypeStruct((ia.shape[0] * CH, LANES), jnp.float32)
                 for ia in idx_arrays]
    mesh = plsc.VectorSubcoreMesh(core_axis_name="c", subcore_axis_name="s")

    @functools.partial(
        pl.kernel, mesh=mesh, out_type=out_types,
        compiler_params=pltpu.CompilerParams(use_tc_tiling_on_sc=False),
        scratch_types=[
            pltpu.VMEM((CHB, 128), jnp.int32),
            pltpu.VMEM((CHB, 128), jnp.int32),
            pltpu.VMEM((CH, LANES), jnp.float32),
            pltpu.VMEM((CH, LANES), jnp.float32),
            pltpu.SemaphoreType.DMA,
            pltpu.SemaphoreType.DMA,
            pltpu.SemaphoreType.DMA,
            pltpu.SemaphoreType.DMA,
        ],
    )
    def sc_kernel(table_ref, *refs):
        n_idx = len(idx_arrays)
        idx_refs = refs[:n_idx]
        out_refs = refs[n_idx:2 * n_idx]
        idx0, idx1, rows0, rows1, g0, g1, o0, o1 = refs[2 * n_idx:]
        wid = lax.axis_index("s") * NC + lax.axis_index("c")
        idx_b = (idx0, idx1)
        rows_b = (rows0, rows1)
        g_sem = (g0, g1)
        o_sem = (o0, o1)

        def start(idx_src, table, chunk, b):
            pltpu.sync_copy(idx_src.at[chunk], idx_b[b])
            for j in range(CHB):
                pltpu.async_copy(table.at[idx_b[b].at[j]],
                                 rows_b[b].at[pl.ds(j * 128, 128)], g_sem[b])

        def finish(table, out, chunk, b):
            for j in range(CHB):
                pltpu.make_async_copy(table.at[idx_b[b].at[j]],
                                      rows_b[b].at[pl.ds(j * 128, 128)],
                                      g_sem[b]).wait()
            pltpu.async_copy(rows_b[b], out.at[pl.ds(chunk * CH, CH)], o_sem[b])

        def drain(out, chunk, b):
            pltpu.make_async_copy(rows_b[b], out.at[pl.ds(chunk * CH, CH)],
                                  o_sem[b]).wait()

        def run_task(idx_src, table, out):
            total = idx_src.shape[0]
            if total < NW:
                @pl.when(wid < total)
                def _():
                    start(idx_src, table, wid, 0)
                    finish(table, out, wid, 0)
                    drain(out, wid, 0)
                return
            n_chunks = total // NW
            c_lo = wid * n_chunks
            if n_chunks == 1:
                start(idx_src, table, c_lo, 0)
                finish(table, out, c_lo, 0)
                drain(out, c_lo, 0)
                return
            n_half = n_chunks // 2

            start(idx_src, table, c_lo, 0)

            def body(c2, _):
                c = c_lo + 2 * c2
                finish(table, out, c, 0)
                start(idx_src, table, c + 1, 1)
                drain(out, c, 0)
                finish(table, out, c + 1, 1)

                @pl.when(c2 + 1 < n_half)
                def _():
                    start(idx_src, table, c + 2, 0)

                drain(out, c + 1, 1)
                return ()

            lax.fori_loop(0, n_half, body, ())

        for idx_ref, out_ref in zip(idx_refs, out_refs):
            run_task(idx_ref, table_ref, out_ref)

    outs = sc_kernel(table, *idx_arrays)
    return outs if isinstance(outs, (tuple, list)) else (outs,)


def _attn(nm, no, q, mask):
    scores = jnp.sum(nm * q[:, None, :], axis=-1)
    scores = jnp.where(mask, scores, jnp.finfo(scores.dtype).min)
    m = jnp.max(scores, axis=1, keepdims=True)
    e = jnp.exp(scores - m)
    p = e / jnp.sum(e, axis=1, keepdims=True)
    return jnp.sum(no * p[:, :, None], axis=1)


def _bf16_dot(a, b):
    return jnp.dot(a.astype(jnp.bfloat16), b.astype(jnp.bfloat16),
                   preferred_element_type=jnp.float32)


def _branch(u, v, nm, no, mask, hop_wT, hop_b, dense_wT, dense_b, out_w):
    q = u + v
    o0 = _attn(nm, no, q, mask)
    q1 = jax.nn.relu(_bf16_dot(q, hop_wT) + o0 + hop_b)
    o1 = _attn(nm, no, q1, mask)
    x = jnp.concatenate([u * v, o1], axis=1)
    h = jax.nn.relu(_bf16_dot(x, dense_wT) + dense_b)
    hb = h.astype(jnp.bfloat16).astype(jnp.float32)
    wb = out_w.astype(jnp.bfloat16).astype(jnp.float32)
    return jnp.sum(hb * wb, axis=1, keepdims=True)


def _cmn_kernel(bb, u_ref, v_ref, vn_ref, nm_ref, no_ref, nmn_ref, non_ref,
                len_ref, lenn_ref, hop_wT_ref, hop_b_ref, dense_wT_ref,
                dense_b_ref, out_w_ref, pos_ref, neg_ref):
    u = u_ref[...]
    hop_wT = hop_wT_ref[...]
    hop_b = hop_b_ref[...]
    dense_wT = dense_wT_ref[...]
    dense_b = dense_b_ref[...]
    out_w = out_w_ref[...]

    def neigh(ref):
        return ref[...].reshape(bb, MAXNP, LANES)

    pos_iota = jax.lax.broadcasted_iota(jnp.int32, (bb, MAXNP), 1)
    mask = pos_iota < len_ref[...]
    mask_n = pos_iota < lenn_ref[...]
    pos_ref[...] = _branch(u, v_ref[...], neigh(nm_ref), neigh(no_ref),
                           mask, hop_wT, hop_b, dense_wT, dense_b, out_w)
    neg_ref[...] = _branch(u, vn_ref[...], neigh(nmn_ref), neigh(non_ref),
                           mask_n, hop_wT, hop_b, dense_wT, dense_b, out_w)


def _pad_weights(hop_w, hop_b, dense_w, dense_b, out_w):
    hop_wT128 = jnp.pad(hop_w.T, ((0, LANES - EMB), (0, LANES - EMB)))
    hop_b128 = jnp.pad(hop_b.reshape(1, EMB), ((0, 0), (0, LANES - EMB)))
    dwT = dense_w.T
    dense_wT256 = jnp.pad(
        dwT.reshape(2, EMB, EMB), ((0, 0), (0, LANES - EMB), (0, LANES - EMB))
    ).reshape(2 * LANES, LANES)
    dense_b128 = jnp.pad(dense_b.reshape(1, EMB), ((0, 0), (0, LANES - EMB)))
    out_w128 = jnp.pad(out_w, ((0, 0), (0, LANES - EMB)))
    return hop_wT128, hop_b128, dense_wT256, dense_b128, out_w128


def _cmn_compute(cur_user, cur_item, cur_item_neg, nm, no, nmn, non,
                 lengths, lengths_n, hop_w, hop_b, dense_w, dense_b, out_w,
                 interpret=False):
    B = cur_user.shape[0]
    Bb = min(128, B)
    grid = (B // Bb,)
    row_spec = pl.BlockSpec((Bb, LANES), lambda i: (i, 0))
    neigh_spec = pl.BlockSpec((Bb * MAXNP, LANES), lambda i: (i, 0))
    len_spec = pl.BlockSpec((Bb, 1), lambda i: (i, 0))
    w_spec = pl.BlockSpec((LANES, LANES), lambda i: (0, 0))
    dw_spec = pl.BlockSpec((2 * LANES, LANES), lambda i: (0, 0))
    b_spec = pl.BlockSpec((1, LANES), lambda i: (0, 0))
    out_spec = pl.BlockSpec((Bb, 1), lambda i: (i, 0))
    hop_wT128, hop_b128, dense_wT256, dense_b128, out_w128 = _pad_weights(
        hop_w, hop_b, dense_w, dense_b, out_w)
    pos, neg = pl.pallas_call(
        functools.partial(_cmn_kernel, Bb),
        grid=grid,
        in_specs=[row_spec, row_spec, row_spec,
                  neigh_spec, neigh_spec, neigh_spec, neigh_spec,
                  len_spec, len_spec, w_spec, b_spec, dw_spec, b_spec, b_spec],
        out_specs=[out_spec, out_spec],
        out_shape=[jax.ShapeDtypeStruct((B, 1), jnp.float32),
                   jax.ShapeDtypeStruct((B, 1), jnp.float32)],
        interpret=interpret,
    )(cur_user, cur_item, cur_item_neg, nm, no, nmn, non,
      lengths.reshape(B, 1), lengths_n.reshape(B, 1),
      hop_wT128, hop_b128, dense_wT256, dense_b128, out_w128)
    return pos[:, 0], neg[:, 0]


NSLICES = 4


def kernel(input_users, input_items, input_items_negative, input_neighborhoods,
           input_neighborhood_lengths, input_neighborhoods_negative,
           input_neighborhood_lengths_negative, user_memory, item_memory,
           user_output, hop_w, hop_b, dense_w, dense_b, out_w):
    B = input_users.shape[0]
    pad = ((0, 0), (0, LANES - EMB))
    um128 = jnp.pad(user_memory, pad)
    im128 = jnp.pad(item_memory, pad)
    uo128 = jnp.pad(user_output, pad)
    Bs = B // NSLICES
    dummy = jnp.arange(Bs * (MAXNP - MAXN), dtype=jnp.int32).reshape(
        Bs, MAXNP - MAXN) % user_memory.shape[0]

    pos_parts, neg_parts = [], []
    for s in range(NSLICES):
        sl = slice(s * Bs, (s + 1) * Bs)
        nidx = jnp.concatenate([input_neighborhoods[sl], dummy], axis=1
                               ).reshape(-1, CHB, 128)
        nnidx = jnp.concatenate([input_neighborhoods_negative[sl], dummy],
                                axis=1).reshape(-1, CHB, 128)
        uidx = input_users[sl].reshape(-1, CHB, 128)
        iidx = input_items[sl].reshape(-1, CHB, 128)
        inidx = input_items_negative[sl].reshape(-1, CHB, 128)
        ci, cin = _sc_gather_from(im128, iidx, inidx)
        nm, nmn, cu = _sc_gather_from(um128, nidx, nnidx, uidx)
        no, non = _sc_gather_from(uo128, nidx, nnidx)
        pos, neg = _cmn_compute(cu, ci, cin, nm, no, nmn, non,
                                input_neighborhood_lengths[sl],
                                input_neighborhood_lengths_negative[sl],
                                hop_w, hop_b, dense_w, dense_b, out_w)
        pos_parts.append(pos)
        neg_parts.append(neg)
    if NSLICES == 1:
        return pos_parts[0], neg_parts[0]
    return (jnp.concatenate(pos_parts), jnp.concatenate(neg_parts))


# speedup vs baseline: 7.3583x; 1.1384x over previous
import functools

import jax
import jax.numpy as jnp
from jax import lax
from jax.experimental import pallas as pl
from jax.experimental.pallas import tpu as pltpu
from jax.experimental.pallas import tpu_sc as plsc

EMB = 64
MAXN = 50
MAXNP = 56
LANES = 128
CH = 256
CHB = CH // 128


def _sc_gather_from(table, *idx_arrays):
    info = plsc.get_sparse_core_info()
    NC, NS = info.num_cores, info.num_subcores
    NW = NC * NS

    out_types = [jax.ShapeDtypeStruct((ia.shape[0] * CH, LANES), jnp.float32)
                 for ia in idx_arrays]
    mesh = plsc.VectorSubcoreMesh(core_axis_name="c", subcore_axis_name="s")

    @functools.partial(
        pl.kernel, mesh=mesh, out_type=out_types,
        compiler_params=pltpu.CompilerParams(use_tc_tiling_on_sc=False),
        scratch_types=[
            pltpu.VMEM((CHB, 128), jnp.int32),
            pltpu.VMEM((CHB, 128), jnp.int32),
            pltpu.VMEM((CH, LANES), jnp.float32),
            pltpu.VMEM((CH, LANES), jnp.float32),
            pltpu.SemaphoreType.DMA,
            pltpu.SemaphoreType.DMA,
            pltpu.SemaphoreType.DMA,
            pltpu.SemaphoreType.DMA,
        ],
    )
    def sc_kernel(table_ref, *refs):
        n_idx = len(idx_arrays)
        idx_refs = refs[:n_idx]
        out_refs = refs[n_idx:2 * n_idx]
        idx0, idx1, rows0, rows1, g0, g1, o0, o1 = refs[2 * n_idx:]
        wid = lax.axis_index("s") * NC + lax.axis_index("c")
        idx_b = (idx0, idx1)
        rows_b = (rows0, rows1)
        g_sem = (g0, g1)
        o_sem = (o0, o1)

        def start(idx_src, table, chunk, b):
            pltpu.sync_copy(idx_src.at[chunk], idx_b[b])
            for j in range(CHB):
                pltpu.async_copy(table.at[idx_b[b].at[j]],
                                 rows_b[b].at[pl.ds(j * 128, 128)], g_sem[b])

        def finish(table, out, chunk, b):
            for j in range(CHB):
                pltpu.make_async_copy(table.at[idx_b[b].at[j]],
                                      rows_b[b].at[pl.ds(j * 128, 128)],
                                      g_sem[b]).wait()
            pltpu.async_copy(rows_b[b], out.at[pl.ds(chunk * CH, CH)], o_sem[b])

        def drain(out, chunk, b):
            pltpu.make_async_copy(rows_b[b], out.at[pl.ds(chunk * CH, CH)],
                                  o_sem[b]).wait()

        def run_task(idx_src, table, out):
            total = idx_src.shape[0]
            if total < NW:
                @pl.when(wid < total)
                def _():
                    start(idx_src, table, wid, 0)
                    finish(table, out, wid, 0)
                    drain(out, wid, 0)
                return
            n_chunks = total // NW
            c_lo = wid * n_chunks
            if n_chunks == 1:
                start(idx_src, table, c_lo, 0)
                finish(table, out, c_lo, 0)
                drain(out, c_lo, 0)
                return
            n_half = n_chunks // 2

            start(idx_src, table, c_lo, 0)

            def body(c2, _):
                c = c_lo + 2 * c2
                finish(table, out, c, 0)
                start(idx_src, table, c + 1, 1)
                drain(out, c, 0)
                finish(table, out, c + 1, 1)

                @pl.when(c2 + 1 < n_half)
                def _():
                    start(idx_src, table, c + 2, 0)

                drain(out, c + 1, 1)
                return ()

            lax.fori_loop(0, n_half, body, ())

        for idx_ref, out_ref in zip(idx_refs, out_refs):
            run_task(idx_ref, table_ref, out_ref)

    outs = sc_kernel(table, *idx_arrays)
    return outs if isinstance(outs, (tuple, list)) else (outs,)


def _tp_kernel(rb, tin_ref, out_ref):
    t = tin_ref[...].T
    out_ref[...] = jnp.concatenate(
        [t, jnp.zeros((rb, LANES - EMB), jnp.float32)], axis=1)


def _transpose_pad(table_t):
    rows = table_t.shape[1]
    rb = 4096
    grid = (pl.cdiv(rows, rb),)
    return pl.pallas_call(
        functools.partial(_tp_kernel, rb),
        grid=grid,
        in_specs=[pl.BlockSpec((EMB, rb), lambda i: (0, i))],
        out_specs=pl.BlockSpec((rb, LANES), lambda i: (i, 0)),
        out_shape=jax.ShapeDtypeStruct((rows, LANES), jnp.float32),
    )(table_t)


def _attn(nm, no, q, mask):
    scores = jnp.sum(nm * q[:, None, :], axis=-1)
    scores = jnp.where(mask, scores, jnp.finfo(scores.dtype).min)
    m = jnp.max(scores, axis=1, keepdims=True)
    e = jnp.exp(scores - m)
    p = e / jnp.sum(e, axis=1, keepdims=True)
    return jnp.sum(no * p[:, :, None], axis=1)


def _bf16_dot(a, b):
    return jnp.dot(a.astype(jnp.bfloat16), b.astype(jnp.bfloat16),
                   preferred_element_type=jnp.float32)


def _branch(u, v, nm, no, mask, hop_wT, hop_b, dense_wT, dense_b, out_w):
    q = u + v
    o0 = _attn(nm, no, q, mask)
    q1 = jax.nn.relu(_bf16_dot(q, hop_wT) + o0 + hop_b)
    o1 = _attn(nm, no, q1, mask)
    x = jnp.concatenate([u * v, o1], axis=1)
    h = jax.nn.relu(_bf16_dot(x, dense_wT) + dense_b)
    hb = h.astype(jnp.bfloat16).astype(jnp.float32)
    wb = out_w.astype(jnp.bfloat16).astype(jnp.float32)
    return jnp.sum(hb * wb, axis=1, keepdims=True)


def _cmn_kernel(bb, u_ref, v_ref, vn_ref, nm_ref, no_ref, nmn_ref, non_ref,
                len_ref, lenn_ref, hop_wT_ref, hop_b_ref, dense_wT_ref,
                dense_b_ref, out_w_ref, pos_ref, neg_ref):
    u = u_ref[...]
    hop_wT = hop_wT_ref[...]
    hop_b = hop_b_ref[...]
    dense_wT = dense_wT_ref[...]
    dense_b = dense_b_ref[...]
    out_w = out_w_ref[...]

    def neigh(ref):
        return ref[...].reshape(bb, MAXNP, LANES)

    pos_iota = jax.lax.broadcasted_iota(jnp.int32, (bb, MAXNP), 1)
    mask = pos_iota < len_ref[...]
    mask_n = pos_iota < lenn_ref[...]
    pos_ref[...] = _branch(u, v_ref[...], neigh(nm_ref), neigh(no_ref),
                           mask, hop_wT, hop_b, dense_wT, dense_b, out_w)
    neg_ref[...] = _branch(u, vn_ref[...], neigh(nmn_ref), neigh(non_ref),
                           mask_n, hop_wT, hop_b, dense_wT, dense_b, out_w)


def _pad_weights(hop_w, hop_b, dense_w, dense_b, out_w):
    hop_wT128 = jnp.pad(hop_w.T, ((0, LANES - EMB), (0, LANES - EMB)))
    hop_b128 = jnp.pad(hop_b.reshape(1, EMB), ((0, 0), (0, LANES - EMB)))
    dwT = dense_w.T
    dense_wT256 = jnp.pad(
        dwT.reshape(2, EMB, EMB), ((0, 0), (0, LANES - EMB), (0, LANES - EMB))
    ).reshape(2 * LANES, LANES)
    dense_b128 = jnp.pad(dense_b.reshape(1, EMB), ((0, 0), (0, LANES - EMB)))
    out_w128 = jnp.pad(out_w, ((0, 0), (0, LANES - EMB)))
    return hop_wT128, hop_b128, dense_wT256, dense_b128, out_w128


def _cmn_compute(cur_user, cur_item, cur_item_neg, nm, no, nmn, non,
                 lengths, lengths_n, hop_w, hop_b, dense_w, dense_b, out_w,
                 interpret=False):
    B = cur_user.shape[0]
    Bb = min(128, B)
    grid = (B // Bb,)
    row_spec = pl.BlockSpec((Bb, LANES), lambda i: (i, 0))
    neigh_spec = pl.BlockSpec((Bb * MAXNP, LANES), lambda i: (i, 0))
    len_spec = pl.BlockSpec((Bb, 1), lambda i: (i, 0))
    w_spec = pl.BlockSpec((LANES, LANES), lambda i: (0, 0))
    dw_spec = pl.BlockSpec((2 * LANES, LANES), lambda i: (0, 0))
    b_spec = pl.BlockSpec((1, LANES), lambda i: (0, 0))
    out_spec = pl.BlockSpec((Bb, 1), lambda i: (i, 0))
    hop_wT128, hop_b128, dense_wT256, dense_b128, out_w128 = _pad_weights(
        hop_w, hop_b, dense_w, dense_b, out_w)
    pos, neg = pl.pallas_call(
        functools.partial(_cmn_kernel, Bb),
        grid=grid,
        in_specs=[row_spec, row_spec, row_spec,
                  neigh_spec, neigh_spec, neigh_spec, neigh_spec,
                  len_spec, len_spec, w_spec, b_spec, dw_spec, b_spec, b_spec],
        out_specs=[out_spec, out_spec],
        out_shape=[jax.ShapeDtypeStruct((B, 1), jnp.float32),
                   jax.ShapeDtypeStruct((B, 1), jnp.float32)],
        interpret=interpret,
    )(cur_user, cur_item, cur_item_neg, nm, no, nmn, non,
      lengths.reshape(B, 1), lengths_n.reshape(B, 1),
      hop_wT128, hop_b128, dense_wT256, dense_b128, out_w128)
    return pos[:, 0], neg[:, 0]


NSLICES = 4


def kernel(input_users, input_items, input_items_negative, input_neighborhoods,
           input_neighborhood_lengths, input_neighborhoods_negative,
           input_neighborhood_lengths_negative, user_memory, item_memory,
           user_output, hop_w, hop_b, dense_w, dense_b, out_w):
    B = input_users.shape[0]
    um128 = _transpose_pad(user_memory.T)
    im128 = _transpose_pad(item_memory.T)
    uo128 = _transpose_pad(user_output.T)
    Bs = B // NSLICES
    dummy = jnp.arange(Bs * (MAXNP - MAXN), dtype=jnp.int32).reshape(
        Bs, MAXNP - MAXN) % user_memory.shape[0]

    pos_parts, neg_parts = [], []
    for s in range(NSLICES):
        sl = slice(s * Bs, (s + 1) * Bs)
        nidx = jnp.concatenate([input_neighborhoods[sl], dummy], axis=1
                               ).reshape(-1, CHB, 128)
        nnidx = jnp.concatenate([input_neighborhoods_negative[sl], dummy],
                                axis=1).reshape(-1, CHB, 128)
        uidx = input_users[sl].reshape(-1, CHB, 128)
        iidx = input_items[sl].reshape(-1, CHB, 128)
        inidx = input_items_negative[sl].reshape(-1, CHB, 128)
        ci, cin = _sc_gather_from(im128, iidx, inidx)
        nm, nmn, cu = _sc_gather_from(um128, nidx, nnidx, uidx)
        no, non = _sc_gather_from(uo128, nidx, nnidx)
        pos, neg = _cmn_compute(cu, ci, cin, nm, no, nmn, non,
                                input_neighborhood_lengths[sl],
                                input_neighborhood_lengths_negative[sl],
                                hop_w, hop_b, dense_w, dense_b, out_w)
        pos_parts.append(pos)
        neg_parts.append(neg)
    if NSLICES == 1:
        return pos_parts[0], neg_parts[0]
    return (jnp.concatenate(pos_parts), jnp.concatenate(neg_parts))
